# baseline (speedup 1.0000x reference)
import jax
import jax.numpy as jnp
from jax.experimental import pallas as pl
from jax.experimental.pallas import tpu as pltpu


def _chunks(seq_len):
    if seq_len % 2 == 0 and seq_len >= 2:
        h = seq_len // 2
        return [(0, h), (h, h)]
    return [(0, seq_len)]


def _make_body(batch, chunks):
    nvm = batch - 1

    def body(emb_ref, out_ref, vmem, read_sems, write_sems, direct_sems):
        for j, (start, rows) in enumerate(chunks):
            sl = pl.ds(start, rows)
            pltpu.make_async_copy(
                emb_ref.at[sl, :], out_ref.at[nvm, sl, :], direct_sems.at[j]
            ).start()
        for j, (start, rows) in enumerate(chunks):
            sl = pl.ds(start, rows)
            pltpu.make_async_copy(
                emb_ref.at[sl, :], vmem.at[sl, :], read_sems.at[j]
            ).start()
        for j, (start, rows) in enumerate(chunks):
            sl = pl.ds(start, rows)
            pltpu.make_async_copy(
                emb_ref.at[sl, :], vmem.at[sl, :], read_sems.at[j]
            ).wait()
            for b in range(nvm):
                pltpu.make_async_copy(
                    vmem.at[sl, :], out_ref.at[b, sl, :], write_sems.at[j, b]
                ).start()
        for j, (start, rows) in enumerate(chunks):
            sl = pl.ds(start, rows)
            for b in range(nvm):
                pltpu.make_async_copy(
                    vmem.at[sl, :], out_ref.at[b, sl, :], write_sems.at[j, b]
                ).wait()
        for j, (start, rows) in enumerate(chunks):
            sl = pl.ds(start, rows)
            pltpu.make_async_copy(
                emb_ref.at[sl, :], out_ref.at[nvm, sl, :], direct_sems.at[j]
            ).wait()

    return body


def kernel(sequence, embeddings):
    batch, seq_len, feat = sequence.shape
    chunks = _chunks(seq_len)

    return pl.pallas_call(
        _make_body(batch, chunks),
        in_specs=[pl.BlockSpec(memory_space=pl.ANY)],
        out_specs=pl.BlockSpec(memory_space=pl.ANY),
        out_shape=jax.ShapeDtypeStruct((batch, seq_len, feat), sequence.dtype),
        scratch_shapes=[
            pltpu.VMEM((seq_len, feat), sequence.dtype),
            pltpu.SemaphoreType.DMA((len(chunks),)),
            pltpu.SemaphoreType.DMA((len(chunks), batch)),
            pltpu.SemaphoreType.DMA((len(chunks),)),
        ],
    )(embeddings)


# speedup vs baseline: 20.7155x; 20.7155x over previous
import jax
import jax.numpy as jnp
from jax.experimental import pallas as pl
from jax.experimental.pallas import tpu as pltpu


def _chunks(seq_len):
    if seq_len % 2 == 0 and seq_len >= 2:
        h = seq_len // 2
        return [(0, h), (h, h)]
    return [(0, seq_len)]


def _make_body(batch, chunks):
    def body(emb_ref, out_ref, vmem, read_sems, write_sems):
        for j, (start, rows) in enumerate(chunks):
            sl = pl.ds(start, rows)
            pltpu.make_async_copy(
                emb_ref.at[sl, :], vmem.at[sl, :], read_sems.at[j]
            ).start()
        for j, (start, rows) in enumerate(chunks):
            sl = pl.ds(start, rows)
            pltpu.make_async_copy(
                emb_ref.at[sl, :], vmem.at[sl, :], read_sems.at[j]
            ).wait()
            for b in range(batch):
                pltpu.make_async_copy(
                    vmem.at[sl, :], out_ref.at[b, sl, :], write_sems.at[j, b]
                ).start()
        for j, (start, rows) in enumerate(chunks):
            sl = pl.ds(start, rows)
            for b in range(batch):
                pltpu.make_async_copy(
                    vmem.at[sl, :], out_ref.at[b, sl, :], write_sems.at[j, b]
                ).wait()

    return body


def kernel(sequence, embeddings):
    batch, seq_len, feat = sequence.shape
    chunks = _chunks(seq_len)

    return pl.pallas_call(
        _make_body(batch, chunks),
        in_specs=[pl.BlockSpec(memory_space=pl.ANY)],
        out_specs=pl.BlockSpec(memory_space=pl.ANY),
        out_shape=jax.ShapeDtypeStruct((batch, seq_len, feat), sequence.dtype),
        scratch_shapes=[
            pltpu.VMEM((seq_len, feat), sequence.dtype),
            pltpu.SemaphoreType.DMA((len(chunks),)),
            pltpu.SemaphoreType.DMA((len(chunks), batch)),
        ],
    )(embeddings)
